# baseline (speedup 1.0000x reference)
import functools

import jax
import jax.numpy as jnp
from jax import lax
from jax.experimental import pallas as pl
from jax.experimental.pallas import tpu as pltpu
from jax.experimental.pallas import tpu_sc as plsc

V = 32000
B = 4096
LABEL_SMOOTHING = 0.1
CONFIDENCE = 1.0 - LABEL_SMOOTHING
IGNORE_COL = (-100) % V
SMOOTH = LABEL_SMOOTHING / (V - 2)

ROWS_PER_BLOCK = 128

_NC = 2
_NS = 16
_NW = _NC * _NS
_CHUNK = B // _NW
_L = 16

_sc_mesh = plsc.VectorSubcoreMesh(core_axis_name="c", subcore_axis_name="s")


@functools.partial(
    pl.kernel,
    mesh=_sc_mesh,
    out_type=jax.ShapeDtypeStruct((B,), jnp.float32),
    scratch_types=[
        pltpu.VMEM((_CHUNK,), jnp.int32),
        pltpu.VMEM((_CHUNK,), jnp.int32),
        pltpu.VMEM((_CHUNK,), jnp.float32),
        pltpu.SemaphoreType.DMA,
    ],
)
def _gather_xt(flat_hbm, tgt_hbm, out_hbm, t_v, idx_v, val_v, sem):
    wid = lax.axis_index("s") * _NC + lax.axis_index("c")
    base = wid * _CHUNK
    pltpu.sync_copy(tgt_hbm.at[pl.ds(base, _CHUNK)], t_v)
    for j in range(_CHUNK // _L):
        t16 = t_v[pl.ds(j * _L, _L)]
        rows = base + j * _L + lax.iota(jnp.int32, _L)
        idx_v[pl.ds(j * _L, _L)] = rows * V + t16
    pltpu.async_copy(flat_hbm.at[idx_v], val_v, sem).wait()
    pltpu.sync_copy(val_v, out_hbm.at[pl.ds(base, _CHUNK)])


def _loss_block_kernel(x_ref, t_ref, xt_ref, out_ref):
    i = pl.program_id(0)
    x = x_ref[...]
    t = t_ref[0, 0, :]
    x_t = xt_ref[0, 0, :]

    m = jnp.max(x, axis=1, keepdims=True)
    se = jnp.sum(jnp.exp(x - m), axis=1)
    lse = m[:, 0] + jnp.log(se)
    sx = jnp.sum(x, axis=1)
    x_i = x[:, IGNORE_COL]

    logp_t = x_t - lse
    logp_i = x_i - lse
    is_i = (t == IGNORE_COL).astype(jnp.float32)

    slog_s = SMOOTH * jnp.log(SMOOTH)
    clog_c = CONFIDENCE * jnp.log(CONFIDENCE)
    base = (V - 2 + is_i) * slog_s + clog_c
    cross = (SMOOTH * (sx - V * lse)
             + (CONFIDENCE - SMOOTH) * logp_t
             - SMOOTH * logp_i
             + is_i * SMOOTH * logp_i)
    partial = jnp.sum(base - cross)

    @pl.when(i == 0)
    def _init():
        out_ref[0, 0] = 0.0

    out_ref[0, 0] += partial


@jax.jit
def kernel(output, target, one_hot):
    del one_hot
    b, v = output.shape
    r = ROWS_PER_BLOCK
    grid = b // r
    tgt = target.astype(jnp.int32)
    xt = _gather_xt(output.reshape(-1), tgt)
    t3 = tgt.reshape(grid, 1, r)
    xt3 = xt.reshape(grid, 1, r)
    total = pl.pallas_call(
        _loss_block_kernel,
        grid=(grid,),
        in_specs=[
            pl.BlockSpec((r, v), lambda i: (i, 0)),
            pl.BlockSpec((1, 1, r), lambda i: (i, 0, 0)),
            pl.BlockSpec((1, 1, r), lambda i: (i, 0, 0)),
        ],
        out_specs=pl.BlockSpec(memory_space=pltpu.SMEM),
        out_shape=jax.ShapeDtypeStruct((1, 1), jnp.float32),
    )(output, t3, xt3)
    return (total[0, 0] / b).astype(jnp.float32)


# speedup vs baseline: 2.5461x; 2.5461x over previous
import jax
import jax.numpy as jnp
from jax import lax
from jax.experimental import pallas as pl
from jax.experimental.pallas import tpu as pltpu

V = 32000
B = 4096
LABEL_SMOOTHING = 0.1
CONFIDENCE = 1.0 - LABEL_SMOOTHING
IGNORE_COL = (-100) % V
SMOOTH = LABEL_SMOOTHING / (V - 2)

ROWS_PER_BLOCK = 64


def _loss_block_kernel(x_ref, t_ref, out_ref):
    i = pl.program_id(0)
    r = ROWS_PER_BLOCK
    x = x_ref[...]
    t = t_ref[0, 0, :]

    m = jnp.max(x, axis=1, keepdims=True)
    se = jnp.sum(jnp.exp(x - m), axis=1)
    lse = m[:, 0] + jnp.log(se)

    col = lax.broadcasted_iota(jnp.int32, (r, V), 1)
    w = jnp.where(col == t[:, None], CONFIDENCE, SMOOTH)
    fused = jnp.sum(x * w, axis=1)

    x_i = x[:, IGNORE_COL]
    is_i = (t == IGNORE_COL).astype(jnp.float32)

    slog_s = SMOOTH * jnp.log(SMOOTH)
    clog_c = CONFIDENCE * jnp.log(CONFIDENCE)
    base = (V - 2 + is_i) * slog_s + clog_c
    cross = fused - (1.0 - is_i) * SMOOTH * x_i - lse * (1.0 + is_i * SMOOTH)
    partial = jnp.sum(base - cross)

    @pl.when(i == 0)
    def _init():
        out_ref[0, 0] = 0.0

    out_ref[0, 0] += partial


@jax.jit
def kernel(output, target, one_hot):
    del one_hot
    b, v = output.shape
    r = ROWS_PER_BLOCK
    grid = b // r
    t3 = target.astype(jnp.int32).reshape(grid, 1, r)
    total = pl.pallas_call(
        _loss_block_kernel,
        grid=(grid,),
        in_specs=[
            pl.BlockSpec((r, v), lambda i: (i, 0)),
            pl.BlockSpec((1, 1, r), lambda i: (i, 0, 0)),
        ],
        out_specs=pl.BlockSpec(memory_space=pltpu.SMEM),
        out_shape=jax.ShapeDtypeStruct((1, 1), jnp.float32),
    )(output, t3)
    return (total[0, 0] / b).astype(jnp.float32)


# speedup vs baseline: 2.8846x; 1.1330x over previous
import jax
import jax.numpy as jnp
from jax import lax
from jax.experimental import pallas as pl
from jax.experimental.pallas import tpu as pltpu

V = 32000
B = 4096
LABEL_SMOOTHING = 0.1
CONFIDENCE = 1.0 - LABEL_SMOOTHING
IGNORE_COL = (-100) % V
SMOOTH = LABEL_SMOOTHING / (V - 2)

ROWS_PER_BLOCK = 128


def _loss_block_kernel(x_ref, t_ref, out_ref):
    i = pl.program_id(0)
    r = ROWS_PER_BLOCK
    x = x_ref[...]
    t = t_ref[0, 0, :]

    m = jnp.max(x, axis=1, keepdims=True)
    se = jnp.sum(jnp.exp(x - m), axis=1)
    lse = m[:, 0] + jnp.log(se)

    col = lax.broadcasted_iota(jnp.int32, (r, V), 1)
    w = jnp.where(col == t[:, None], CONFIDENCE, SMOOTH)
    fused = jnp.sum(x * w, axis=1)

    x_i = x[:, IGNORE_COL]
    is_i = (t == IGNORE_COL).astype(jnp.float32)

    slog_s = SMOOTH * jnp.log(SMOOTH)
    clog_c = CONFIDENCE * jnp.log(CONFIDENCE)
    base = (V - 2 + is_i) * slog_s + clog_c
    cross = fused - (1.0 - is_i) * SMOOTH * x_i - lse * (1.0 + is_i * SMOOTH)
    partial = jnp.sum(base - cross)

    @pl.when(i == 0)
    def _init():
        out_ref[0, 0] = 0.0

    out_ref[0, 0] += partial


@jax.jit
def kernel(output, target, one_hot):
    del one_hot
    b, v = output.shape
    r = ROWS_PER_BLOCK
    grid = b // r
    t3 = target.astype(jnp.int32).reshape(grid, 1, r)
    total = pl.pallas_call(
        _loss_block_kernel,
        grid=(grid,),
        in_specs=[
            pl.BlockSpec((r, v), lambda i: (i, 0)),
            pl.BlockSpec((1, 1, r), lambda i: (i, 0, 0)),
        ],
        out_specs=pl.BlockSpec(memory_space=pltpu.SMEM),
        out_shape=jax.ShapeDtypeStruct((1, 1), jnp.float32),
    )(output, t3)
    return (total[0, 0] / b).astype(jnp.float32)
